# baseline (speedup 1.0000x reference)
import functools

import numpy as np
import jax
import jax.numpy as jnp
from jax import lax
from jax.experimental import pallas as pl
from jax.experimental.pallas import tpu as pltpu
from jax.experimental.pallas import tpu_sc as plsc

DIM = 96
PE = 96
NH = 8
HD = DIM // NH
SCALE = HD ** -0.5
NB, CC, HH, WW = 2, 96, 224, 224
HW = HH * WW
NHW = NB * HW
NV = 4
RB = 8
PROWS = 8
PBLK = PROWS * WW
GCH = 128


def _np_pb():
    npf = PE // 2
    sc = 2 * np.pi
    ones = np.ones((1, 2, 2), np.float64)
    y_embed = np.cumsum(ones, axis=1)
    x_embed = np.cumsum(ones, axis=2)
    eps = 1e-6
    y_embed = y_embed / (y_embed[:, -1:, :] + eps) * sc
    x_embed = x_embed / (x_embed[:, :, -1:] + eps) * sc
    dim_t = 10000.0 ** (2 * (np.arange(npf) // 2) / npf)
    pos_x = x_embed[..., None] / dim_t
    pos_y = y_embed[..., None] / dim_t
    pos_x = np.stack((np.sin(pos_x[..., 0::2]), np.cos(pos_x[..., 1::2])),
                     axis=4).reshape(1, 2, 2, npf)
    pos_y = np.stack((np.sin(pos_y[..., 0::2]), np.cos(pos_y[..., 1::2])),
                     axis=4).reshape(1, 2, 2, npf)
    pos = np.concatenate((pos_y, pos_x), axis=3)
    return pos.reshape(NV, PE).astype(np.float32)


_PB_CONST = _np_pb()
_POSY0 = np.zeros((PE // 2,), np.float32)
_POSY0[1::2] = 1.0
_HMAT = np.zeros((DIM, NH), np.float32)
for _d in range(DIM):
    _HMAT[_d, _d // HD] = 1.0
_PDEG = 13


def _np_pe_poly():
    invd = (2.0 * np.pi / (2.0 + 1e-6)) / (
        10000.0 ** (np.arange(PE // 4) * 2.0 / (PE // 2)))
    g = np.linspace(0.0, 1.0, 4096)
    v = g[:, None] ** np.arange(_PDEG)[None, :]
    a = g[:, None] * invd[None, :]
    peg = np.empty((g.size, PE // 2))
    peg[:, 0::2] = np.sin(a)
    peg[:, 1::2] = np.cos(a)
    m, _, _, _ = np.linalg.lstsq(v, peg, rcond=None)
    return m.astype(np.float32)


_PEPOLY = _np_pe_poly()


def _p1_body(ym, yt, yb_, xm, xt_in, xb_, wy, wx, cb, idx_ref, frac_ref,
             y128_ref, xt_ref):
    ni = pl.program_id(0)
    hi = pl.program_id(1)
    nhb = HH // RB
    tmask = (hi > 0).astype(jnp.float32)
    bmask = (hi < nhb - 1).astype(jnp.float32)
    ty = [jnp.transpose(yt[0, :, RB - 1, :]) * tmask]
    ty += [jnp.transpose(ym[0, :, r, :]) for r in range(RB)]
    ty += [jnp.transpose(yb_[0, :, 0, :]) * bmask]
    tx = [jnp.transpose(xt_in[0, :, RB - 1, :]) * tmask]
    tx += [jnp.transpose(xm[0, :, r, :]) for r in range(RB)]
    tx += [jnp.transpose(xb_[0, :, 0, :]) * bmask]
    y128_ref[0] = jnp.stack(ty[1:1 + RB], axis=0)
    xt_ref[0] = jnp.stack(tx[1:1 + RB], axis=0)
    bf = jnp.bfloat16
    f32 = jnp.float32
    zc = jnp.zeros((RB + 2, 1, CC), jnp.float32)
    rows_y = jnp.concatenate(
        [zc, jnp.stack(ty, axis=0), zc], axis=1).astype(bf).astype(f32)
    rows_x = jnp.concatenate(
        [zc, jnp.stack(tx, axis=0), zc], axis=1).astype(bf).astype(f32)
    wyb = wy[...].astype(bf).astype(f32)
    wxb = wx[...].astype(bf).astype(f32)
    acc = jnp.zeros((RB, WW, CC), jnp.float32)
    for di in range(3):
        for dj in range(3):
            t = di * 3 + dj
            acc = acc + rows_y[di:di + RB, dj:dj + WW, :] * wyb[t]
            acc = acc + rows_x[di:di + RB, dj:dj + WW, :] * wxb[t]
    off = jnp.sum(acc, axis=-1) + cb[...]
    offs = off * 2.0 / WW
    jf = lax.broadcasted_iota(jnp.int32, (RB, WW), 1).astype(jnp.float32)
    gcol = jf + offs
    c0 = jnp.floor(gcol)
    frac_ref[0] = gcol - c0
    ii = hi * RB + lax.broadcasted_iota(jnp.int32, (RB, WW), 0)
    base = ni * HW
    for j in range(NV):
        dy, dx = j // 2, j % 2
        rr = jnp.minimum(ii + dy, HH - 1)
        ccj = jnp.clip(c0 + dx, 0.0, float(WW - 1)).astype(jnp.int32)
        idx_ref[j, 0] = base + rr * WW + ccj


def _p1(y, x, wy, wx, cb):
    nhb = HH // RB
    in_specs = []
    for _ in range(2):
        in_specs.append(pl.BlockSpec((1, CC, RB, WW),
                                     lambda ni, hi: (ni, 0, hi, 0)))
        in_specs.append(pl.BlockSpec(
            (1, CC, RB, WW),
            lambda ni, hi: (ni, 0, jnp.maximum(hi - 1, 0), 0)))
        in_specs.append(pl.BlockSpec(
            (1, CC, RB, WW),
            lambda ni, hi: (ni, 0, jnp.minimum(hi + 1, nhb - 1), 0)))
    in_specs += [
        pl.BlockSpec((9, CC), lambda ni, hi: (0, 0)),
        pl.BlockSpec((9, CC), lambda ni, hi: (0, 0)),
        pl.BlockSpec((1, WW), lambda ni, hi: (0, 0)),
    ]
    return pl.pallas_call(
        _p1_body,
        grid=(NB, nhb),
        in_specs=in_specs,
        out_specs=[
            pl.BlockSpec((NV, 1, RB, WW), lambda ni, hi: (0, ni, hi, 0)),
            pl.BlockSpec((1, RB, WW), lambda ni, hi: (ni, hi, 0)),
            pl.BlockSpec((1, RB, WW, CC), lambda ni, hi: (ni, hi, 0, 0)),
            pl.BlockSpec((1, RB, WW, CC), lambda ni, hi: (ni, hi, 0, 0)),
        ],
        out_shape=[
            jax.ShapeDtypeStruct((NV, NB, HH, WW), jnp.int32),
            jax.ShapeDtypeStruct((NB, HH, WW), jnp.float32),
            jax.ShapeDtypeStruct((NB, HH, WW, CC), jnp.float32),
            jax.ShapeDtypeStruct((NB, HH, WW, CC), jnp.float32),
        ],
    )(y, y, y, x, x, x, wy, wx, cb)


def _p2_gather(y_t, idx_flat):
    info = plsc.get_sparse_core_info()
    nw = info.num_cores * info.num_subcores
    total = NV * NHW
    per_w = total // nw
    n_ch = per_w // GCH
    mesh = plsc.VectorSubcoreMesh(core_axis_name="c", subcore_axis_name="s")

    @functools.partial(
        pl.kernel,
        out_type=jax.ShapeDtypeStruct((total, CC), jnp.float32),
        mesh=mesh,
        scratch_types=[
            pltpu.VMEM((GCH,), jnp.int32),
            pltpu.VMEM((GCH, CC), jnp.float32),
            pltpu.SemaphoreType.DMA,
        ],
        compiler_params=pltpu.CompilerParams(use_tc_tiling_on_sc=False),
    )
    def gath(table_hbm, idx_hbm, out_hbm, idx_v, rows_v, sem):
        wid = lax.axis_index("s") * info.num_cores + lax.axis_index("c")
        w0 = wid * per_w

        def body(t, carry):
            bs = w0 + t * GCH
            pltpu.sync_copy(idx_hbm.at[pl.ds(bs, GCH)], idx_v)
            pltpu.async_copy(table_hbm.at[idx_v], rows_v, sem).wait()
            pltpu.sync_copy(rows_v, out_hbm.at[pl.ds(bs, GCH)])
            return carry

        lax.fori_loop(0, n_ch, body, 0)

    return gath(y_t, idx_flat)


def _p3_body(xt, frac, wrp, Wq, Wk, Wv, Cp, qc, kb, vb, Hm, Ht, out_ref):
    f32 = jnp.float32
    q = jnp.dot(xt[...], Wq[...], preferred_element_type=f32)
    fr = frac[0]
    rows = [jnp.ones_like(fr), fr]
    for _ in range(_PDEG - 2):
        rows.append(rows[-1] * fr)
    fp = jnp.concatenate(rows, axis=0)
    q = q + lax.dot_general(fp, Cp[...], (((0,), (0,)), ((), ())),
                            preferred_element_type=f32)
    q = (q + qc[...]) * SCALE
    wall = wrp[...].reshape(NV * PBLK, CC)
    kk = jnp.dot(wall, Wk[...], preferred_element_type=f32)
    vv = jnp.dot(wall, Wv[...], preferred_element_type=f32)
    kk3 = kk.reshape(NV, PBLK, DIM) + kb[...][:, None, :]
    qk = kk3 * q[None, :, :]
    l3 = jnp.dot(qk.reshape(NV * PBLK, DIM), Hm[...],
                 preferred_element_type=f32).reshape(NV, PBLK, NH)
    m = jnp.maximum(jnp.maximum(l3[0], l3[1]), jnp.maximum(l3[2], l3[3]))
    es = jnp.exp(l3 - m[None, :, :])
    den = es[0] + es[1] + es[2] + es[3]
    aw = es / den[None, :, :]
    af = jnp.dot(aw.reshape(NV * PBLK, NH), Ht[...],
                 preferred_element_type=f32).reshape(NV, PBLK, DIM)
    vv3 = vv.reshape(NV, PBLK, DIM) + vb[...][:, None, :]
    av = af * vv3
    o = av[0] + av[1] + av[2] + av[3]
    for r in range(PROWS):
        out_ref[0, :, r, :] = jnp.transpose(o[r * WW:(r + 1) * WW])


def _p3(x_t, frac3, wrp3, Wq, Wk, Wv, Cp, qc, kb, vb, Hm, Ht):
    nhb = HH // PROWS
    full = lambda shape: pl.BlockSpec(shape,
                                      lambda ni, hb: tuple(0 for _ in shape))
    return pl.pallas_call(
        _p3_body,
        grid=(NB, nhb),
        in_specs=[
            pl.BlockSpec((PBLK, CC), lambda ni, hb: (ni * nhb + hb, 0)),
            pl.BlockSpec((1, 1, PBLK), lambda ni, hb: (ni * nhb + hb, 0, 0)),
            pl.BlockSpec((NV, PBLK, CC),
                         lambda ni, hb: (0, ni * nhb + hb, 0)),
            full((PE, DIM)), full((PE, DIM)), full((PE, DIM)),
            full((_PDEG, DIM)),
            full((1, DIM)), full((NV, DIM)), full((NV, DIM)),
            full((DIM, NH)), full((NH, DIM)),
        ],
        out_specs=pl.BlockSpec((1, DIM, PROWS, WW),
                               lambda ni, hb: (ni, 0, hb, 0)),
        out_shape=jax.ShapeDtypeStruct((NB, DIM, HH, WW), jnp.float32),
    )(x_t, frac3, wrp3, Wq, Wk, Wv, Cp, qc, kb, vb, Hm, Ht)


def kernel(y, x, Wq, bq, Wk, bk, Wv, bv, conv_w, conv_b):
    wy = conv_w[0, :CC].transpose(1, 2, 0).reshape(9, CC)
    wx = conv_w[0, CC:].transpose(1, 2, 0).reshape(9, CC)
    cb = jnp.broadcast_to(conv_b.reshape(1, 1), (1, WW))

    idx, frac, y128, x_t = _p1(y, x, wy, wx, cb)

    wrp = _p2_gather(y128.reshape(NHW, CC), idx.reshape(NV * NHW))
    wrp3 = wrp.reshape(NV, NHW, CC)

    pbc = jnp.asarray(_PB_CONST)
    kb = pbc @ Wk + bk
    vb = pbc @ Wv + bv
    qc = (jnp.asarray(_POSY0) @ Wq[:PE // 2] + bq).reshape(1, DIM)
    Cp = jnp.asarray(_PEPOLY) @ Wq[PE // 2:]
    return _p3(x_t.reshape(NHW, CC), frac.reshape(NHW // PBLK, 1, PBLK),
               wrp3, Wq, Wk, Wv, Cp, qc, kb, vb,
               jnp.asarray(_HMAT), jnp.asarray(_HMAT.T))


# speedup vs baseline: 1.5033x; 1.5033x over previous
import functools

import numpy as np
import jax
import jax.numpy as jnp
from jax import lax
from jax.experimental import pallas as pl
from jax.experimental.pallas import tpu as pltpu
from jax.experimental.pallas import tpu_sc as plsc

DIM = 96
PE = 96
NH = 8
HD = DIM // NH
SCALE = HD ** -0.5
NB, CC, HH, WW = 2, 96, 224, 224
HW = HH * WW
NHW = NB * HW
NV = 4
RB = 8
PROWS = 8
PBLK = PROWS * WW
GCH = 128


def _np_pb():
    npf = PE // 2
    sc = 2 * np.pi
    ones = np.ones((1, 2, 2), np.float64)
    y_embed = np.cumsum(ones, axis=1)
    x_embed = np.cumsum(ones, axis=2)
    eps = 1e-6
    y_embed = y_embed / (y_embed[:, -1:, :] + eps) * sc
    x_embed = x_embed / (x_embed[:, :, -1:] + eps) * sc
    dim_t = 10000.0 ** (2 * (np.arange(npf) // 2) / npf)
    pos_x = x_embed[..., None] / dim_t
    pos_y = y_embed[..., None] / dim_t
    pos_x = np.stack((np.sin(pos_x[..., 0::2]), np.cos(pos_x[..., 1::2])),
                     axis=4).reshape(1, 2, 2, npf)
    pos_y = np.stack((np.sin(pos_y[..., 0::2]), np.cos(pos_y[..., 1::2])),
                     axis=4).reshape(1, 2, 2, npf)
    pos = np.concatenate((pos_y, pos_x), axis=3)
    return pos.reshape(NV, PE).astype(np.float32)


_PB_CONST = _np_pb()
_POSY0 = np.zeros((PE // 2,), np.float32)
_POSY0[1::2] = 1.0
_HMAT = np.zeros((DIM, NH), np.float32)
for _d in range(DIM):
    _HMAT[_d, _d // HD] = 1.0
_PDEG = 13


def _np_pe_poly():
    invd = (2.0 * np.pi / (2.0 + 1e-6)) / (
        10000.0 ** (np.arange(PE // 4) * 2.0 / (PE // 2)))
    g = np.linspace(0.0, 1.0, 4096)
    v = g[:, None] ** np.arange(_PDEG)[None, :]
    a = g[:, None] * invd[None, :]
    peg = np.empty((g.size, PE // 2))
    peg[:, 0::2] = np.sin(a)
    peg[:, 1::2] = np.cos(a)
    m, _, _, _ = np.linalg.lstsq(v, peg, rcond=None)
    return m.astype(np.float32)


_PEPOLY = _np_pe_poly()


def _p1_body(ym, yt, yb_, xm, xt_in, xb_, wy, wx, cb, idx_ref, frac_ref,
             y128_ref, xt_ref):
    ni = pl.program_id(0)
    hi = pl.program_id(1)
    nhb = HH // RB
    tmask = (hi > 0).astype(jnp.float32)
    bmask = (hi < nhb - 1).astype(jnp.float32)
    ty = [jnp.transpose(yt[0, :, RB - 1, :]) * tmask]
    ty += [jnp.transpose(ym[0, :, r, :]) for r in range(RB)]
    ty += [jnp.transpose(yb_[0, :, 0, :]) * bmask]
    tx = [jnp.transpose(xt_in[0, :, RB - 1, :]) * tmask]
    tx += [jnp.transpose(xm[0, :, r, :]) for r in range(RB)]
    tx += [jnp.transpose(xb_[0, :, 0, :]) * bmask]
    yblk = jnp.stack(ty[1:1 + RB], axis=0)
    y128_ref[0] = jnp.pad(yblk, ((0, 0), (0, 0), (0, 128 - CC)))
    xt_ref[0] = jnp.stack(tx[1:1 + RB], axis=0)
    bf = jnp.bfloat16
    f32 = jnp.float32
    zc = jnp.zeros((RB + 2, 1, CC), jnp.float32)
    rows_y = jnp.concatenate(
        [zc, jnp.stack(ty, axis=0), zc], axis=1).astype(bf).astype(f32)
    rows_x = jnp.concatenate(
        [zc, jnp.stack(tx, axis=0), zc], axis=1).astype(bf).astype(f32)
    wyb = wy[...].astype(bf).astype(f32)
    wxb = wx[...].astype(bf).astype(f32)
    acc = jnp.zeros((RB, WW, CC), jnp.float32)
    for di in range(3):
        for dj in range(3):
            t = di * 3 + dj
            acc = acc + rows_y[di:di + RB, dj:dj + WW, :] * wyb[t]
            acc = acc + rows_x[di:di + RB, dj:dj + WW, :] * wxb[t]
    off = jnp.sum(acc, axis=-1) + cb[...]
    offs = off * 2.0 / WW
    jf = lax.broadcasted_iota(jnp.int32, (RB, WW), 1).astype(jnp.float32)
    gcol = jf + offs
    c0 = jnp.floor(gcol)
    frac_ref[0] = gcol - c0
    ii = hi * RB + lax.broadcasted_iota(jnp.int32, (RB, WW), 0)
    base = ni * HW
    for j in range(NV):
        dy, dx = j // 2, j % 2
        rr = jnp.minimum(ii + dy, HH - 1)
        ccj = jnp.clip(c0 + dx, 0.0, float(WW - 1)).astype(jnp.int32)
        idx_ref[j, 0] = base + rr * WW + ccj


def _p1(y, x, wy, wx, cb):
    nhb = HH // RB
    in_specs = []
    for _ in range(2):
        in_specs.append(pl.BlockSpec((1, CC, RB, WW),
                                     lambda ni, hi: (ni, 0, hi, 0)))
        in_specs.append(pl.BlockSpec(
            (1, CC, RB, WW),
            lambda ni, hi: (ni, 0, jnp.maximum(hi - 1, 0), 0)))
        in_specs.append(pl.BlockSpec(
            (1, CC, RB, WW),
            lambda ni, hi: (ni, 0, jnp.minimum(hi + 1, nhb - 1), 0)))
    in_specs += [
        pl.BlockSpec((9, CC), lambda ni, hi: (0, 0)),
        pl.BlockSpec((9, CC), lambda ni, hi: (0, 0)),
        pl.BlockSpec((1, WW), lambda ni, hi: (0, 0)),
    ]
    return pl.pallas_call(
        _p1_body,
        grid=(NB, nhb),
        in_specs=in_specs,
        out_specs=[
            pl.BlockSpec((NV, 1, RB, WW), lambda ni, hi: (0, ni, hi, 0)),
            pl.BlockSpec((1, RB, WW), lambda ni, hi: (ni, hi, 0)),
            pl.BlockSpec((1, RB, WW, 128), lambda ni, hi: (ni, hi, 0, 0)),
            pl.BlockSpec((1, RB, WW, CC), lambda ni, hi: (ni, hi, 0, 0)),
        ],
        out_shape=[
            jax.ShapeDtypeStruct((NV, NB, HH, WW), jnp.int32),
            jax.ShapeDtypeStruct((NB, HH, WW), jnp.float32),
            jax.ShapeDtypeStruct((NB, HH, WW, 128), jnp.float32),
            jax.ShapeDtypeStruct((NB, HH, WW, CC), jnp.float32),
        ],
    )(y, y, y, x, x, x, wy, wx, cb)


def _p2_gather(y_t, idx_flat):
    info = plsc.get_sparse_core_info()
    nw = info.num_cores * info.num_subcores
    total = NV * NHW
    per_w = total // nw
    n_ch = per_w // GCH
    mesh = plsc.VectorSubcoreMesh(core_axis_name="c", subcore_axis_name="s")

    @functools.partial(
        pl.kernel,
        out_type=jax.ShapeDtypeStruct((total, 128), jnp.float32),
        mesh=mesh,
        scratch_types=[
            pltpu.VMEM((GCH,), jnp.int32),
            pltpu.VMEM((GCH,), jnp.int32),
            pltpu.VMEM((GCH, 128), jnp.float32),
            pltpu.VMEM((GCH, 128), jnp.float32),
            pltpu.SemaphoreType.DMA,
            pltpu.SemaphoreType.DMA,
        ],
    )
    def gath(table_hbm, idx_hbm, out_hbm, i0, i1, r0, r1, s0, s1):
        wid = lax.axis_index("s") * info.num_cores + lax.axis_index("c")
        w0 = wid * per_w

        def wait_g(rv, sv):
            pltpu.make_async_copy(table_hbm.at[pl.ds(0, GCH)], rv, sv).wait()

        pltpu.sync_copy(idx_hbm.at[pl.ds(w0, GCH)], i0)
        pltpu.async_copy(table_hbm.at[i0], r0, s0)

        def body(t2, carry):
            p = w0 + (2 * t2) * GCH
            pltpu.sync_copy(idx_hbm.at[pl.ds(p + GCH, GCH)], i1)
            pltpu.async_copy(table_hbm.at[i1], r1, s1)
            wait_g(r0, s0)
            pltpu.sync_copy(r0, out_hbm.at[pl.ds(p, GCH)])
            pltpu.sync_copy(idx_hbm.at[pl.ds(p + 2 * GCH, GCH)], i0)
            pltpu.async_copy(table_hbm.at[i0], r0, s0)
            wait_g(r1, s1)
            pltpu.sync_copy(r1, out_hbm.at[pl.ds(p + GCH, GCH)])
            return carry

        lax.fori_loop(0, n_ch // 2 - 1, body, 0)
        last = w0 + (n_ch - 1) * GCH
        pltpu.sync_copy(idx_hbm.at[pl.ds(last, GCH)], i1)
        pltpu.async_copy(table_hbm.at[i1], r1, s1)
        wait_g(r0, s0)
        pltpu.sync_copy(r0, out_hbm.at[pl.ds(last - GCH, GCH)])
        wait_g(r1, s1)
        pltpu.sync_copy(r1, out_hbm.at[pl.ds(last, GCH)])

    return gath(y_t, idx_flat)


def _p3_body(xt, frac, wrp, Wq, Wk, Wv, Cp, qc, kb, vb, Hm, Ht, out_ref):
    f32 = jnp.float32
    q = jnp.dot(xt[...], Wq[...], preferred_element_type=f32)
    fr = frac[0]
    rows = [jnp.ones_like(fr), fr]
    for _ in range(_PDEG - 2):
        rows.append(rows[-1] * fr)
    fp = jnp.concatenate(rows, axis=0)
    q = q + lax.dot_general(fp, Cp[...], (((0,), (0,)), ((), ())),
                            preferred_element_type=f32)
    q = (q + qc[...]) * SCALE
    wall = wrp[...].reshape(NV * PBLK, 128)
    kk = jnp.dot(wall, Wk[...], preferred_element_type=f32)
    vv = jnp.dot(wall, Wv[...], preferred_element_type=f32)
    kk3 = kk.reshape(NV, PBLK, DIM) + kb[...][:, None, :]
    qk = kk3 * q[None, :, :]
    l3 = jnp.dot(qk.reshape(NV * PBLK, DIM), Hm[...],
                 preferred_element_type=f32).reshape(NV, PBLK, NH)
    m = jnp.maximum(jnp.maximum(l3[0], l3[1]), jnp.maximum(l3[2], l3[3]))
    es = jnp.exp(l3 - m[None, :, :])
    den = es[0] + es[1] + es[2] + es[3]
    aw = es / den[None, :, :]
    af = jnp.dot(aw.reshape(NV * PBLK, NH), Ht[...],
                 preferred_element_type=f32).reshape(NV, PBLK, DIM)
    vv3 = vv.reshape(NV, PBLK, DIM) + vb[...][:, None, :]
    av = af * vv3
    o = av[0] + av[1] + av[2] + av[3]
    for r in range(PROWS):
        out_ref[0, :, r, :] = jnp.transpose(o[r * WW:(r + 1) * WW])


def _p3(x_t, frac3, wrp3, Wq, Wk, Wv, Cp, qc, kb, vb, Hm, Ht):
    nhb = HH // PROWS
    full = lambda shape: pl.BlockSpec(shape,
                                      lambda ni, hb: tuple(0 for _ in shape))
    return pl.pallas_call(
        _p3_body,
        grid=(NB, nhb),
        in_specs=[
            pl.BlockSpec((PBLK, CC), lambda ni, hb: (ni * nhb + hb, 0)),
            pl.BlockSpec((1, 1, PBLK), lambda ni, hb: (ni * nhb + hb, 0, 0)),
            pl.BlockSpec((NV, PBLK, 128),
                         lambda ni, hb: (0, ni * nhb + hb, 0)),
            full((PE, DIM)), full((128, DIM)), full((128, DIM)),
            full((_PDEG, DIM)),
            full((1, DIM)), full((NV, DIM)), full((NV, DIM)),
            full((DIM, NH)), full((NH, DIM)),
        ],
        out_specs=pl.BlockSpec((1, DIM, PROWS, WW),
                               lambda ni, hb: (ni, 0, hb, 0)),
        out_shape=jax.ShapeDtypeStruct((NB, DIM, HH, WW), jnp.float32),
    )(x_t, frac3, wrp3, Wq, Wk, Wv, Cp, qc, kb, vb, Hm, Ht)


def kernel(y, x, Wq, bq, Wk, bk, Wv, bv, conv_w, conv_b):
    wy = conv_w[0, :CC].transpose(1, 2, 0).reshape(9, CC)
    wx = conv_w[0, CC:].transpose(1, 2, 0).reshape(9, CC)
    cb = jnp.broadcast_to(conv_b.reshape(1, 1), (1, WW))

    idx, frac, y128, x_t = _p1(y, x, wy, wx, cb)

    wrp = _p2_gather(y128.reshape(NHW, 128), idx.reshape(NV * NHW))
    wrp3 = wrp.reshape(NV, NHW, 128)

    pbc = jnp.asarray(_PB_CONST)
    kb = pbc @ Wk + bk
    vb = pbc @ Wv + bv
    qc = (jnp.asarray(_POSY0) @ Wq[:PE // 2] + bq).reshape(1, DIM)
    Cp = jnp.asarray(_PEPOLY) @ Wq[PE // 2:]
    Wk128 = jnp.pad(Wk, ((0, 128 - PE), (0, 0)))
    Wv128 = jnp.pad(Wv, ((0, 128 - PE), (0, 0)))

    return _p3(x_t.reshape(NHW, CC), frac.reshape(NHW // PBLK, 1, PBLK),
               wrp3, Wq, Wk128, Wv128, Cp, qc, kb, vb,
               jnp.asarray(_HMAT), jnp.asarray(_HMAT.T))
